# baseline (speedup 1.0000x reference)
import functools

import jax
import jax.numpy as jnp
from jax import lax
from jax.experimental import pallas as pl
from jax.experimental.pallas import tpu as pltpu
from jax.experimental.pallas import tpu_sc as plsc

NDIM = 16
NKNOT = 64
NSEG = NKNOT - 1

NC = 2
NS = 16
NW = NC * NS
LANES = 16

NDATA = 1_000_000
NFULLCOL = NDATA // 128
NTAIL = NDATA - NFULLCOL * 128
COL_BASE = NFULLCOL // NW
COL_REM = NFULLCOL % NW
SLAB = 8
SLAB_W = SLAB * 128

B_S = NDIM * NKNOT
B_B = 2 * NDIM * NKNOT
B_L = 3 * NDIM * NKNOT


def _tables_body(x0_ref, y0_ref, ldx_ref, ldy_ref, xt_ref,
                 out_ref, yt_ref, lt_ref):
    ldx = ldx_ref[...]
    ldy = ldy_ref[...]
    dx = jnp.exp(ldx)
    dy = jnp.exp(ldy)
    row = lax.broadcasted_iota(jnp.int32, (NSEG, NKNOT), 0)
    col = lax.broadcasted_iota(jnp.int32, (NSEG, NKNOT), 1)
    tri = (row < col).astype(jnp.float32)
    xx = x0_ref[...] + jnp.dot(dx, tri, preferred_element_type=jnp.float32)
    yy = y0_ref[...] + jnp.dot(dy, tri, preferred_element_type=jnp.float32)
    s = (yy[:, 1:NKNOT] - yy[:, 0:NSEG]) / (xx[:, 1:NKNOT] - xx[:, 0:NSEG])
    ls = jnp.log(s)
    b = yy[:, 0:NSEG] - s * xx[:, 0:NSEG]
    inf2 = jnp.full((NDIM, 2), jnp.inf, jnp.float32)
    pad1 = jnp.zeros((NDIM, 1), jnp.float32)
    out_ref[...] = jnp.concatenate(
        [
            jnp.concatenate([xx[:, 1:NSEG], inf2], axis=1),
            jnp.concatenate([s, pad1 + 1.0], axis=1),
            jnp.concatenate([b, pad1], axis=1),
            jnp.concatenate([ls, pad1], axis=1),
        ],
        axis=0,
    )
    xt = xt_ref[...]
    acc_s = jnp.broadcast_to(s[:, 0:1], (NDIM, NTAIL))
    acc_b = jnp.broadcast_to(b[:, 0:1], (NDIM, NTAIL))
    acc_l = jnp.broadcast_to(ls[:, 0:1], (NDIM, NTAIL))
    for k in range(1, NSEG):
        m = (xx[:, k:k + 1] < xt).astype(jnp.float32)
        acc_s = acc_s + (s[:, k:k + 1] - s[:, k - 1:k]) * m
        acc_b = acc_b + (b[:, k:k + 1] - b[:, k - 1:k]) * m
        acc_l = acc_l + (ls[:, k:k + 1] - ls[:, k - 1:k]) * m
    yt_ref[...] = acc_s * xt + acc_b
    lt_ref[...] = acc_l


def _build_tables(x0, y0, logdx, logdy, x_tail_t):
    return pl.pallas_call(
        _tables_body,
        out_shape=(
            jax.ShapeDtypeStruct((4 * NDIM, NKNOT), jnp.float32),
            jax.ShapeDtypeStruct((NDIM, NTAIL), jnp.float32),
            jax.ShapeDtypeStruct((NDIM, NTAIL), jnp.float32),
        ),
    )(x0, y0, logdx, logdy, x_tail_t)


def _spline_vecs(tab_v, x_v, y_v, ls_v, d, nvec, unroll):
    bt = jnp.full((LANES,), d * NKNOT, jnp.int32)

    @plsc.parallel_loop(0, nvec, unroll=unroll)
    def _vec(j):
        v = x_v[d, pl.ds(j * LANES, LANES)]
        c = bt
        for w in (32, 16, 8, 4, 2, 1):
            probe = plsc.load_gather(tab_v, [c + (w - 1)])
            c = jnp.where(probe < v, c + w, c)
        s = plsc.load_gather(tab_v, [c + B_S])
        b = plsc.load_gather(tab_v, [c + B_B])
        l = plsc.load_gather(tab_v, [c + B_L])
        y_v[d, pl.ds(j * LANES, LANES)] = s * v + b
        ls_v[d, pl.ds(j * LANES, LANES)] = l


NSLAB = COL_BASE // SLAB


def _sc_body(x_hbm, tab_hbm, y_hbm, ls_hbm,
             tab_v, x_v0, x_v1, y_v0, y_v1, l_v0, l_v1,
             sx0, sx1, so0, so1):
    cid = lax.axis_index("c")
    sid = lax.axis_index("s")
    wid = sid * NC + cid
    ncols = jnp.where(wid < COL_REM, COL_BASE + 1, COL_BASE)
    col0 = wid * COL_BASE + jnp.minimum(wid, COL_REM)
    nrem = ncols - NSLAB * SLAB

    xbuf = (x_v0, x_v1)
    ybuf = (y_v0, y_v1)
    lbuf = (l_v0, l_v1)
    sx = (sx0, sx1)
    so = (so0, so1)

    pltpu.sync_copy(tab_hbm, tab_v)

    def x_slab(sl):
        return x_hbm.at[:, pl.ds((col0 + sl * SLAB) * 128, SLAB_W)]

    pltpu.async_copy(x_slab(0), x_v0, sx0)

    @pl.loop(0, NSLAB, step=2)
    def _slab(s):
        for b in (0, 1):
            sl = s + b
            off = (col0 + sl * SLAB) * 128

            @pl.when(sl + 1 < NSLAB)
            def _prefetch():
                pltpu.async_copy(x_slab(sl + 1), xbuf[1 - b], sx[1 - b])

            pltpu.make_async_copy(x_slab(sl), xbuf[b], sx[b]).wait()

            @pl.when(sl >= 2)
            def _drain_prev_out():
                pltpu.make_async_copy(
                    ybuf[b], y_hbm.at[:, pl.ds(off, SLAB_W)], so[b]).wait()
                pltpu.make_async_copy(
                    lbuf[b], ls_hbm.at[:, pl.ds(off, SLAB_W)], so[b]).wait()

            @pl.loop(0, NDIM)
            def _dim(d):
                _spline_vecs(tab_v, xbuf[b], ybuf[b], lbuf[b], d,
                             SLAB_W // LANES, 16)

            pltpu.async_copy(ybuf[b], y_hbm.at[:, pl.ds(off, SLAB_W)], so[b])
            pltpu.async_copy(lbuf[b], ls_hbm.at[:, pl.ds(off, SLAB_W)], so[b])

    for b in (0, 1):
        pltpu.make_async_copy(
            ybuf[b], y_hbm.at[:, pl.ds(0, SLAB_W)], so[b]).wait()
        pltpu.make_async_copy(
            lbuf[b], ls_hbm.at[:, pl.ds(0, SLAB_W)], so[b]).wait()

    @pl.loop(0, nrem)
    def _col(r):
        off = (col0 + NSLAB * SLAB + r) * 128
        pltpu.sync_copy(x_hbm.at[:, pl.ds(off, 128)], x_v0.at[:, pl.ds(0, 128)])

        @pl.loop(0, NDIM)
        def _dim(d):
            _spline_vecs(tab_v, x_v0, y_v0, l_v0, d, 128 // LANES, 8)

        pltpu.sync_copy(y_v0.at[:, pl.ds(0, 128)], y_hbm.at[:, pl.ds(off, 128)])
        pltpu.sync_copy(l_v0.at[:, pl.ds(0, 128)], ls_hbm.at[:, pl.ds(off, 128)])


@functools.lru_cache(maxsize=1)
def _sc_spline():
    return pl.kernel(
        _sc_body,
        out_type=(
            jax.ShapeDtypeStruct((NDIM, NDATA), jnp.float32),
            jax.ShapeDtypeStruct((NDIM, NDATA), jnp.float32),
        ),
        mesh=plsc.VectorSubcoreMesh(core_axis_name="c", subcore_axis_name="s"),
        compiler_params=pltpu.CompilerParams(needs_layout_passes=False),
        scratch_types=[
            pltpu.VMEM((4 * NDIM * NKNOT,), jnp.float32),
            pltpu.VMEM((NDIM, SLAB_W), jnp.float32),
            pltpu.VMEM((NDIM, SLAB_W), jnp.float32),
            pltpu.VMEM((NDIM, SLAB_W), jnp.float32),
            pltpu.VMEM((NDIM, SLAB_W), jnp.float32),
            pltpu.VMEM((NDIM, SLAB_W), jnp.float32),
            pltpu.VMEM((NDIM, SLAB_W), jnp.float32),
            pltpu.SemaphoreType.DMA,
            pltpu.SemaphoreType.DMA,
            pltpu.SemaphoreType.DMA,
            pltpu.SemaphoreType.DMA,
        ],
    )


def kernel(x, x0, y0, logdx, logdy):
    xt = x.T
    x_tail_t = lax.slice(xt, (0, NFULLCOL * 128), (NDIM, NDATA))
    tab, y_tail_t, ls_tail_t = _build_tables(x0, y0, logdx, logdy, x_tail_t)
    yt, lst = _sc_spline()(xt, tab.reshape(-1))
    y = lax.dynamic_update_slice(yt.T, y_tail_t.T, (NFULLCOL * 128, 0))
    ld = lax.dynamic_update_slice(lst.T, ls_tail_t.T, (NFULLCOL * 128, 0))
    return y, ld


# speedup vs baseline: 1.4239x; 1.4239x over previous
import functools

import jax
import jax.numpy as jnp
from jax import lax
from jax.experimental import pallas as pl
from jax.experimental.pallas import tpu as pltpu
from jax.experimental.pallas import tpu_sc as plsc

NDIM = 16
NKNOT = 64
NSEG = NKNOT - 1

NC = 2
NS = 16
NW = NC * NS
LANES = 16

NDATA = 1_000_000
NFULLCOL = NDATA // 128
NTAIL = NDATA - NFULLCOL * 128
COL_BASE = NFULLCOL // NW
COL_REM = NFULLCOL % NW
SLAB = 8
SLAB_W = SLAB * 128

B_S = NDIM * NKNOT
B_B = 2 * NDIM * NKNOT
B_L = 3 * NDIM * NKNOT


def _tables_body(x0_ref, y0_ref, ldx_ref, ldy_ref, xt_ref,
                 out_ref, yt_ref, lt_ref):
    ldx = ldx_ref[...]
    ldy = ldy_ref[...]
    dx = jnp.exp(ldx)
    dy = jnp.exp(ldy)
    row = lax.broadcasted_iota(jnp.int32, (NSEG, NKNOT), 0)
    col = lax.broadcasted_iota(jnp.int32, (NSEG, NKNOT), 1)
    tri = (row < col).astype(jnp.float32)
    xx = x0_ref[...] + jnp.dot(dx, tri, preferred_element_type=jnp.float32)
    yy = y0_ref[...] + jnp.dot(dy, tri, preferred_element_type=jnp.float32)
    s = (yy[:, 1:NKNOT] - yy[:, 0:NSEG]) / (xx[:, 1:NKNOT] - xx[:, 0:NSEG])
    ls = jnp.log(s)
    b = yy[:, 0:NSEG] - s * xx[:, 0:NSEG]
    inf2 = jnp.full((NDIM, 2), jnp.inf, jnp.float32)
    pad1 = jnp.zeros((NDIM, 1), jnp.float32)
    out_ref[...] = jnp.concatenate(
        [
            jnp.concatenate([xx[:, 1:NSEG], inf2], axis=1),
            jnp.concatenate([s, pad1 + 1.0], axis=1),
            jnp.concatenate([b, pad1], axis=1),
            jnp.concatenate([ls, pad1], axis=1),
        ],
        axis=0,
    )
    xt = xt_ref[...]
    acc_s = jnp.broadcast_to(s[:, 0:1], (NDIM, NTAIL))
    acc_b = jnp.broadcast_to(b[:, 0:1], (NDIM, NTAIL))
    acc_l = jnp.broadcast_to(ls[:, 0:1], (NDIM, NTAIL))
    for k in range(1, NSEG):
        m = (xx[:, k:k + 1] < xt).astype(jnp.float32)
        acc_s = acc_s + (s[:, k:k + 1] - s[:, k - 1:k]) * m
        acc_b = acc_b + (b[:, k:k + 1] - b[:, k - 1:k]) * m
        acc_l = acc_l + (ls[:, k:k + 1] - ls[:, k - 1:k]) * m
    yt_ref[...] = acc_s * xt + acc_b
    lt_ref[...] = acc_l


def _build_tables(x0, y0, logdx, logdy, x_tail_t):
    return pl.pallas_call(
        _tables_body,
        out_shape=(
            jax.ShapeDtypeStruct((4 * NDIM, NKNOT), jnp.float32),
            jax.ShapeDtypeStruct((NDIM, NTAIL), jnp.float32),
            jax.ShapeDtypeStruct((NDIM, NTAIL), jnp.float32),
        ),
    )(x0, y0, logdx, logdy, x_tail_t)


def _spline_vecs(tab_v, x_v, y_v, ls_v, d, nvec, unroll):
    bt = jnp.full((LANES,), d * NKNOT, jnp.int32)

    @plsc.parallel_loop(0, nvec, unroll=unroll)
    def _vec(j):
        v = x_v[d, pl.ds(j * LANES, LANES)]
        c = bt
        for w in (32, 16, 8, 4, 2, 1):
            probe = plsc.load_gather(tab_v, [c + (w - 1)])
            c = jnp.where(probe < v, c + w, c)
        s = plsc.load_gather(tab_v, [c + B_S])
        b = plsc.load_gather(tab_v, [c + B_B])
        l = plsc.load_gather(tab_v, [c + B_L])
        y_v[d, pl.ds(j * LANES, LANES)] = s * v + b
        ls_v[d, pl.ds(j * LANES, LANES)] = l


NSLAB = COL_BASE // SLAB


def _sc_body(x_hbm, tab_hbm, y_hbm, ls_hbm,
             tab_v, x_v0, x_v1, y_v0, y_v1, l_v0, l_v1,
             sx0, sx1, so0, so1):
    cid = lax.axis_index("c")
    sid = lax.axis_index("s")
    wid = sid * NC + cid
    ncols = jnp.where(wid < COL_REM, COL_BASE + 1, COL_BASE)
    col0 = wid * COL_BASE + jnp.minimum(wid, COL_REM)
    nrem = ncols - NSLAB * SLAB

    xbuf = (x_v0, x_v1)
    ybuf = (y_v0, y_v1)
    lbuf = (l_v0, l_v1)
    sx = (sx0, sx1)
    so = (so0, so1)

    pltpu.sync_copy(tab_hbm, tab_v)

    def x_slab(sl):
        return x_hbm.at[:, pl.ds((col0 + sl * SLAB) * 128, SLAB_W)]

    pltpu.async_copy(x_slab(0), x_v0, sx0)

    @pl.loop(0, NSLAB, step=2)
    def _slab(s):
        for b in (0, 1):
            sl = s + b
            off = (col0 + sl * SLAB) * 128

            @pl.when(sl + 1 < NSLAB)
            def _prefetch():
                pltpu.async_copy(x_slab(sl + 1), xbuf[1 - b], sx[1 - b])

            pltpu.make_async_copy(x_slab(sl), xbuf[b], sx[b]).wait()

            @pl.when(sl >= 2)
            def _drain_prev_out():
                pltpu.make_async_copy(
                    ybuf[b], y_hbm.at[:, pl.ds(off, SLAB_W)], so[b]).wait()
                pltpu.make_async_copy(
                    lbuf[b], ls_hbm.at[:, pl.ds(off, SLAB_W)], so[b]).wait()

            @pl.loop(0, NDIM)
            def _dim(d):
                _spline_vecs(tab_v, xbuf[b], ybuf[b], lbuf[b], d,
                             SLAB_W // LANES, 4)

            pltpu.async_copy(ybuf[b], y_hbm.at[:, pl.ds(off, SLAB_W)], so[b])
            pltpu.async_copy(lbuf[b], ls_hbm.at[:, pl.ds(off, SLAB_W)], so[b])

    for b in (0, 1):
        pltpu.make_async_copy(
            ybuf[b], y_hbm.at[:, pl.ds(0, SLAB_W)], so[b]).wait()
        pltpu.make_async_copy(
            lbuf[b], ls_hbm.at[:, pl.ds(0, SLAB_W)], so[b]).wait()

    @pl.loop(0, nrem)
    def _col(r):
        off = (col0 + NSLAB * SLAB + r) * 128
        pltpu.sync_copy(x_hbm.at[:, pl.ds(off, 128)], x_v0.at[:, pl.ds(0, 128)])

        @pl.loop(0, NDIM)
        def _dim(d):
            _spline_vecs(tab_v, x_v0, y_v0, l_v0, d, 128 // LANES, 8)

        pltpu.sync_copy(y_v0.at[:, pl.ds(0, 128)], y_hbm.at[:, pl.ds(off, 128)])
        pltpu.sync_copy(l_v0.at[:, pl.ds(0, 128)], ls_hbm.at[:, pl.ds(off, 128)])


@functools.lru_cache(maxsize=1)
def _sc_spline():
    return pl.kernel(
        _sc_body,
        out_type=(
            jax.ShapeDtypeStruct((NDIM, NDATA), jnp.float32),
            jax.ShapeDtypeStruct((NDIM, NDATA), jnp.float32),
        ),
        mesh=plsc.VectorSubcoreMesh(core_axis_name="c", subcore_axis_name="s"),
        compiler_params=pltpu.CompilerParams(needs_layout_passes=False),
        scratch_types=[
            pltpu.VMEM((4 * NDIM * NKNOT,), jnp.float32),
            pltpu.VMEM((NDIM, SLAB_W), jnp.float32),
            pltpu.VMEM((NDIM, SLAB_W), jnp.float32),
            pltpu.VMEM((NDIM, SLAB_W), jnp.float32),
            pltpu.VMEM((NDIM, SLAB_W), jnp.float32),
            pltpu.VMEM((NDIM, SLAB_W), jnp.float32),
            pltpu.VMEM((NDIM, SLAB_W), jnp.float32),
            pltpu.SemaphoreType.DMA,
            pltpu.SemaphoreType.DMA,
            pltpu.SemaphoreType.DMA,
            pltpu.SemaphoreType.DMA,
        ],
    )


def kernel(x, x0, y0, logdx, logdy):
    xt = x.T
    x_tail_t = lax.slice(xt, (0, NFULLCOL * 128), (NDIM, NDATA))
    tab, y_tail_t, ls_tail_t = _build_tables(x0, y0, logdx, logdy, x_tail_t)
    yt, lst = _sc_spline()(xt, tab.reshape(-1))
    y = lax.dynamic_update_slice(yt.T, y_tail_t.T, (NFULLCOL * 128, 0))
    ld = lax.dynamic_update_slice(lst.T, ls_tail_t.T, (NFULLCOL * 128, 0))
    return y, ld


# speedup vs baseline: 1.4786x; 1.0384x over previous
import functools

import jax
import jax.numpy as jnp
from jax import lax
from jax.experimental import pallas as pl
from jax.experimental.pallas import tpu as pltpu
from jax.experimental.pallas import tpu_sc as plsc

NDIM = 16
NKNOT = 64
NSEG = NKNOT - 1

NC = 2
NS = 16
NW = NC * NS
LANES = 16

NDATA = 1_000_000
NFULLCOL = NDATA // 128
NTAIL = NDATA - NFULLCOL * 128
COL_BASE = NFULLCOL // NW
COL_REM = NFULLCOL % NW
SLAB = 8
SLAB_W = SLAB * 128

B_S = NDIM * NKNOT
B_B = 2 * NDIM * NKNOT
B_L = 3 * NDIM * NKNOT


def _tables_body(x0_ref, y0_ref, ldx_ref, ldy_ref, xt_ref,
                 out_ref, yt_ref, lt_ref):
    ldx = ldx_ref[...]
    ldy = ldy_ref[...]
    dx = jnp.exp(ldx)
    dy = jnp.exp(ldy)
    row = lax.broadcasted_iota(jnp.int32, (NSEG, NKNOT), 0)
    col = lax.broadcasted_iota(jnp.int32, (NSEG, NKNOT), 1)
    tri = (row < col).astype(jnp.float32)
    xx = x0_ref[...] + jnp.dot(dx, tri, preferred_element_type=jnp.float32)
    yy = y0_ref[...] + jnp.dot(dy, tri, preferred_element_type=jnp.float32)
    s = (yy[:, 1:NKNOT] - yy[:, 0:NSEG]) / (xx[:, 1:NKNOT] - xx[:, 0:NSEG])
    ls = jnp.log(s)
    b = yy[:, 0:NSEG] - s * xx[:, 0:NSEG]
    inf2 = jnp.full((NDIM, 2), jnp.inf, jnp.float32)
    pad1 = jnp.zeros((NDIM, 1), jnp.float32)
    out_ref[...] = jnp.concatenate(
        [
            jnp.concatenate([xx[:, 1:NSEG], inf2], axis=1),
            jnp.concatenate([s, pad1 + 1.0], axis=1),
            jnp.concatenate([b, pad1], axis=1),
            jnp.concatenate([ls, pad1], axis=1),
        ],
        axis=0,
    )
    xt = xt_ref[...]
    acc_s = jnp.broadcast_to(s[:, 0:1], (NDIM, NTAIL))
    acc_b = jnp.broadcast_to(b[:, 0:1], (NDIM, NTAIL))
    acc_l = jnp.broadcast_to(ls[:, 0:1], (NDIM, NTAIL))
    for k in range(1, NSEG):
        m = (xx[:, k:k + 1] < xt).astype(jnp.float32)
        acc_s = acc_s + (s[:, k:k + 1] - s[:, k - 1:k]) * m
        acc_b = acc_b + (b[:, k:k + 1] - b[:, k - 1:k]) * m
        acc_l = acc_l + (ls[:, k:k + 1] - ls[:, k - 1:k]) * m
    yt_ref[...] = acc_s * xt + acc_b
    lt_ref[...] = acc_l


def _build_tables(x0, y0, logdx, logdy, x_tail_t):
    return pl.pallas_call(
        _tables_body,
        out_shape=(
            jax.ShapeDtypeStruct((4 * NDIM, NKNOT), jnp.float32),
            jax.ShapeDtypeStruct((NDIM, NTAIL), jnp.float32),
            jax.ShapeDtypeStruct((NDIM, NTAIL), jnp.float32),
        ),
    )(x0, y0, logdx, logdy, x_tail_t)


def _spline_vecs(tab_v, x_v, y_v, ls_v, d, nvec, unroll):
    bt = jnp.full((LANES,), d * NKNOT, jnp.int32)

    @plsc.parallel_loop(0, nvec, unroll=unroll)
    def _vec(j):
        v = x_v[d, pl.ds(j * LANES, LANES)]
        c = bt
        for w in (32, 16, 8, 4, 2, 1):
            probe = plsc.load_gather(tab_v, [c + (w - 1)])
            c = jnp.where(probe < v, c + w, c)
        s = plsc.load_gather(tab_v, [c + B_S])
        b = plsc.load_gather(tab_v, [c + B_B])
        l = plsc.load_gather(tab_v, [c + B_L])
        y_v[d, pl.ds(j * LANES, LANES)] = s * v + b
        ls_v[d, pl.ds(j * LANES, LANES)] = l


NSLAB = COL_BASE // SLAB


def _sc_body(x_hbm, tab_hbm, y_hbm, ls_hbm,
             tab_v, x_v0, x_v1, y_v0, y_v1, l_v0, l_v1,
             sx0, sx1, so0, so1):
    cid = lax.axis_index("c")
    sid = lax.axis_index("s")
    wid = sid * NC + cid
    ncols = jnp.where(wid < COL_REM, COL_BASE + 1, COL_BASE)
    col0 = wid * COL_BASE + jnp.minimum(wid, COL_REM)
    nrem = ncols - NSLAB * SLAB

    xbuf = (x_v0, x_v1)
    ybuf = (y_v0, y_v1)
    lbuf = (l_v0, l_v1)
    sx = (sx0, sx1)
    so = (so0, so1)

    pltpu.sync_copy(tab_hbm, tab_v)

    def x_slab(sl):
        return x_hbm.at[:, pl.ds((col0 + sl * SLAB) * 128, SLAB_W)]

    pltpu.async_copy(x_slab(0), x_v0, sx0)

    @pl.loop(0, NSLAB, step=2)
    def _slab(s):
        for b in (0, 1):
            sl = s + b
            off = (col0 + sl * SLAB) * 128

            @pl.when(sl + 1 < NSLAB)
            def _prefetch():
                pltpu.async_copy(x_slab(sl + 1), xbuf[1 - b], sx[1 - b])

            pltpu.make_async_copy(x_slab(sl), xbuf[b], sx[b]).wait()

            @pl.when(sl >= 2)
            def _drain_prev_out():
                pltpu.make_async_copy(
                    ybuf[b], y_hbm.at[:, pl.ds(off, SLAB_W)], so[b]).wait()
                pltpu.make_async_copy(
                    lbuf[b], ls_hbm.at[:, pl.ds(off, SLAB_W)], so[b]).wait()

            @pl.loop(0, NDIM)
            def _dim(d):
                _spline_vecs(tab_v, xbuf[b], ybuf[b], lbuf[b], d,
                             SLAB_W // LANES, 8)

            pltpu.async_copy(ybuf[b], y_hbm.at[:, pl.ds(off, SLAB_W)], so[b])
            pltpu.async_copy(lbuf[b], ls_hbm.at[:, pl.ds(off, SLAB_W)], so[b])

    for b in (0, 1):
        pltpu.make_async_copy(
            ybuf[b], y_hbm.at[:, pl.ds(0, SLAB_W)], so[b]).wait()
        pltpu.make_async_copy(
            lbuf[b], ls_hbm.at[:, pl.ds(0, SLAB_W)], so[b]).wait()

    @pl.loop(0, nrem)
    def _col(r):
        off = (col0 + NSLAB * SLAB + r) * 128
        pltpu.sync_copy(x_hbm.at[:, pl.ds(off, 128)], x_v0.at[:, pl.ds(0, 128)])

        @pl.loop(0, NDIM)
        def _dim(d):
            _spline_vecs(tab_v, x_v0, y_v0, l_v0, d, 128 // LANES, 8)

        pltpu.sync_copy(y_v0.at[:, pl.ds(0, 128)], y_hbm.at[:, pl.ds(off, 128)])
        pltpu.sync_copy(l_v0.at[:, pl.ds(0, 128)], ls_hbm.at[:, pl.ds(off, 128)])


@functools.lru_cache(maxsize=1)
def _sc_spline():
    return pl.kernel(
        _sc_body,
        out_type=(
            jax.ShapeDtypeStruct((NDIM, NDATA), jnp.float32),
            jax.ShapeDtypeStruct((NDIM, NDATA), jnp.float32),
        ),
        mesh=plsc.VectorSubcoreMesh(core_axis_name="c", subcore_axis_name="s"),
        compiler_params=pltpu.CompilerParams(needs_layout_passes=False),
        scratch_types=[
            pltpu.VMEM((4 * NDIM * NKNOT,), jnp.float32),
            pltpu.VMEM((NDIM, SLAB_W), jnp.float32),
            pltpu.VMEM((NDIM, SLAB_W), jnp.float32),
            pltpu.VMEM((NDIM, SLAB_W), jnp.float32),
            pltpu.VMEM((NDIM, SLAB_W), jnp.float32),
            pltpu.VMEM((NDIM, SLAB_W), jnp.float32),
            pltpu.VMEM((NDIM, SLAB_W), jnp.float32),
            pltpu.SemaphoreType.DMA,
            pltpu.SemaphoreType.DMA,
            pltpu.SemaphoreType.DMA,
            pltpu.SemaphoreType.DMA,
        ],
    )


def kernel(x, x0, y0, logdx, logdy):
    xt = x.T
    x_tail_t = lax.slice(xt, (0, NFULLCOL * 128), (NDIM, NDATA))
    tab, y_tail_t, ls_tail_t = _build_tables(x0, y0, logdx, logdy, x_tail_t)
    yt, lst = _sc_spline()(xt, tab.reshape(-1))
    y = lax.dynamic_update_slice(yt.T, y_tail_t.T, (NFULLCOL * 128, 0))
    ld = lax.dynamic_update_slice(lst.T, ls_tail_t.T, (NFULLCOL * 128, 0))
    return y, ld
